# baseline (speedup 1.0000x reference)
import functools

import jax
import jax.numpy as jnp
from jax import lax
from jax.experimental import pallas as pl
from jax.experimental.pallas import tpu as pltpu
from jax.experimental.pallas import tpu_sc as plsc

N = 10000
NT = 2500
D = 256
K = 16

NP = 10240
NTP = 2560
SPLITS = (512, 1024, 768, 256)
TB = 256
NCH = NP // 128
NG = NCH // 16
_PAD_POS = 1e18

_NCORES = 2
_NSUB = 16
_NW = _NCORES * _NSUB

_F32_INF = float("inf")
_I32_BIG = 2**30

_GDN = lax.GatherDimensionNumbers(
    offset_dims=(), collapsed_slice_dims=(0,), start_index_map=(0,))


def _perm(v, idx):
    return lax.gather(v, idx[:, None], _GDN, slice_sizes=(1,),
                      mode=lax.GatherScatterMode.PROMISE_IN_BOUNDS)


def _splat_min(v, lane):
    for sh in (8, 4, 2, 1):
        v = jnp.minimum(v, _perm(v, lane ^ sh))
    return v


def _splat_max(v, lane):
    for sh in (8, 4, 2, 1):
        v = jnp.maximum(v, _perm(v, lane ^ sh))
    return v


def _tree8(vals, op):
    return op(op(op(vals[0], vals[1]), op(vals[2], vals[3])),
              op(op(vals[4], vals[5]), op(vals[6], vals[7])))


def _dist_body(pt_ref, ps_ref, d2_ref, aux_ref):
    t0 = pt_ref[:, 0:1]
    t1 = pt_ref[:, 1:2]
    t2 = pt_ref[:, 2:3]
    e0 = t0 - ps_ref[0:1, :]
    e1 = t1 - ps_ref[1:2, :]
    e2 = t2 - ps_ref[2:3, :]
    d2 = e0 * e0 + e1 * e1 + e2 * e2
    d2_ref[...] = d2
    aux_ref[...] = jnp.min(d2.reshape(TB, NCH, 128), axis=2)


def _dist_and_chunkmins(pt_pad, ps_pad, rows):
    return pl.pallas_call(
        _dist_body,
        grid=(rows // TB,),
        in_specs=[
            pl.BlockSpec((TB, 3), lambda i: (i, 0)),
            pl.BlockSpec((3, NP), lambda i: (0, 0)),
        ],
        out_specs=[
            pl.BlockSpec((TB, NP), lambda i: (i, 0)),
            pl.BlockSpec((TB, NCH), lambda i: (i, 0)),
        ],
        out_shape=[
            jax.ShapeDtypeStruct((rows, NP), jnp.float32),
            jax.ShapeDtypeStruct((rows, NCH), jnp.float32),
        ],
    )(pt_pad, ps_pad)


TBM = 64


def _max_body(g_ref, o_ref):
    o_ref[...] = jnp.max(g_ref[...], axis=1)


def _max_over_k(gath, rows):
    return pl.pallas_call(
        _max_body,
        grid=(rows // TBM,),
        in_specs=[pl.BlockSpec((TBM, K, D), lambda i: (i, 0, 0))],
        out_specs=pl.BlockSpec((TBM, D), lambda i: (i, 0)),
        out_shape=jax.ShapeDtypeStruct((rows, D), jnp.float32),
    )(gath)


_SC_CACHE = {}


def _get_sc_select_gather(rows):
    if rows in _SC_CACHE:
        return _SC_CACHE[rows]
    RW = rows // _NW

    @functools.partial(
        pl.kernel,
        out_type=jax.ShapeDtypeStruct((rows, K, D), jnp.float32),
        mesh=plsc.VectorSubcoreMesh(core_axis_name="c", subcore_axis_name="s"),
        scratch_types=[
            pltpu.VMEM((RW, NCH), jnp.float32),
            pltpu.VMEM((NP,), jnp.float32),
            pltpu.VMEM((NP,), jnp.float32),
            pltpu.VMEM((K, D), jnp.float32),
            pltpu.VMEM((K, D), jnp.float32),
            pltpu.SemaphoreType.DMA,
            pltpu.SemaphoreType.DMA,
            pltpu.SemaphoreType.DMA,
            pltpu.SemaphoreType.DMA,
            pltpu.SemaphoreType.DMA,
            pltpu.SemaphoreType.DMA,
        ],
    )
    def _sc_body(x_hbm, d2_hbm, aux_hbm, gath_hbm,
                 aux_v, d2a_v, d2b_v, rxa_v, rxb_v,
                 sema, semb, semxa, semxb, semoa, semob):
        wid = lax.axis_index("s") * _NCORES + lax.axis_index("c")
        base = wid * RW
        pltpu.sync_copy(aux_hbm.at[pl.ds(base, RW)], aux_v)
        lane = lax.iota(jnp.int32, 16)

        chid = [None] * NG

        def _round(st, j):
            r, d2r_v, mvs, sel = st
            macc = jnp.minimum(jnp.minimum(jnp.minimum(mvs[0], mvs[1]),
                                           jnp.minimum(mvs[2], mvs[3])),
                               mvs[4])
            msp = _splat_min(macc, lane)
            cacc = jnp.where(mvs[0] == msp, chid[0], _I32_BIG)
            for g in range(1, NG):
                cacc = jnp.minimum(
                    cacc, jnp.where(mvs[g] == msp, chid[g], _I32_BIG))
            csp = _splat_min(cacc, lane)
            cbase = csp[0] * 128
            cps, cxs, sms = [], [], []
            for s in range(8):
                dv = d2r_v[pl.ds(cbase + s * 16, 16)]
                cols = cbase + s * 16 + lane
                iseq = dv == msp
                cps.append(jnp.where(iseq, cols, _I32_BIG))
                cxs.append(jnp.where(iseq, cols, -1))
                sms.append(jnp.where(dv > msp, dv, _F32_INF))
            colsp = _splat_min(_tree8(cps, jnp.minimum), lane)
            cmxsp = _splat_max(_tree8(cxs, jnp.maximum), lane)
            smsp = _splat_min(_tree8(sms, jnp.minimum), lane)
            csel = colsp[0]
            sel = jnp.where(lane == j, colsp, sel)
            soff = (csel // 16) * 16
            dvv = d2r_v[pl.ds(soff, 16)]
            d2r_v[pl.ds(soff, 16)] = jnp.where(
                soff + lane == colsp, _F32_INF, dvv)
            nsp = jnp.where(cmxsp > colsp, msp, smsp)
            mvs = [jnp.where(chid[g] == csp, nsp, mvs[g])
                   for g in range(NG)]
            return (r, d2r_v, mvs, sel)

        def select_rows2(r0, d2e_v, r1, d2o_v):
            for g in range(NG):
                chid[g] = lane + g * 16
            sts = []
            for r, dref in ((r0, d2e_v), (r1, d2o_v)):
                mvs = [aux_v[r, pl.ds(g * 16, 16)] for g in range(NG)]
                sel = jnp.zeros((16,), jnp.int32)
                sts.append((r, dref, mvs, sel))
            for j in range(K):
                sts = [_round(st, j) for st in sts]
            return sts[0][3], sts[1][3]

        pltpu.async_copy(d2_hbm.at[base], d2a_v, sema)
        pltpu.async_copy(d2_hbm.at[base + 1], d2b_v, semb)

        def pair_body(i, carry):
            r0 = i * 2

            @pl.when(i > 0)
            def _():
                pltpu.make_async_copy(x_hbm.at[lane], rxa_v, semxa).wait()
                pltpu.async_copy(rxa_v, gath_hbm.at[base + r0 - 2], semoa)
                pltpu.make_async_copy(x_hbm.at[lane], rxb_v, semxb).wait()
                pltpu.async_copy(rxb_v, gath_hbm.at[base + r0 - 1], semob)

            pltpu.make_async_copy(d2_hbm.at[base], d2a_v, sema).wait()
            pltpu.make_async_copy(d2_hbm.at[base], d2b_v, semb).wait()
            sel_e, sel_o = select_rows2(r0, d2a_v, r0 + 1, d2b_v)

            @pl.when(r0 + 2 < RW)
            def _():
                pltpu.async_copy(d2_hbm.at[base + r0 + 2], d2a_v, sema)

            @pl.when(r0 + 3 < RW)
            def _():
                pltpu.async_copy(d2_hbm.at[base + r0 + 3], d2b_v, semb)

            @pl.when(i > 0)
            def _():
                pltpu.make_async_copy(rxa_v, gath_hbm.at[base], semoa).wait()
                pltpu.make_async_copy(rxb_v, gath_hbm.at[base], semob).wait()
            pltpu.async_copy(x_hbm.at[sel_e], rxa_v, semxa)
            pltpu.async_copy(x_hbm.at[sel_o], rxb_v, semxb)
            return carry

        lax.fori_loop(0, RW // 2, pair_body, 0)
        pltpu.make_async_copy(x_hbm.at[lane], rxa_v, semxa).wait()
        pltpu.sync_copy(rxa_v, gath_hbm.at[base + RW - 2])
        pltpu.make_async_copy(x_hbm.at[lane], rxb_v, semxb).wait()
        pltpu.sync_copy(rxb_v, gath_hbm.at[base + RW - 1])

    _SC_CACHE[rows] = _sc_body
    return _sc_body


def kernel(x, pos, batch_x, pos_target, batch_target, k):
    pt_pad = jnp.zeros((NTP, 3), jnp.float32).at[:NT].set(pos_target)
    ps_pad = jnp.full((3, NP), _PAD_POS, jnp.float32).at[:, :N].set(pos.T)
    outs = []
    r0 = 0
    for rows in SPLITS:
        d2h, aux = _dist_and_chunkmins(pt_pad[r0:r0 + rows], ps_pad, rows)
        gath = _get_sc_select_gather(rows)(x, d2h, aux)
        outs.append(_max_over_k(gath, rows))
        r0 += rows
    return jnp.concatenate(outs, axis=0)[:NT]


# speedup vs baseline: 1.0333x; 1.0333x over previous
import functools

import jax
import jax.numpy as jnp
from jax import lax
from jax.experimental import pallas as pl
from jax.experimental.pallas import tpu as pltpu
from jax.experimental.pallas import tpu_sc as plsc

N = 10000
NT = 2500
D = 256
K = 16

NP = 10240
NTP = 2560
SPLITS = (512, 1024, 1024)
TB = 256
NCH = NP // 128
NG = NCH // 16
_PAD_POS = 1e18

_NCORES = 2
_NSUB = 16
_NW = _NCORES * _NSUB

_F32_INF = float("inf")
_I32_BIG = 2**30

_GDN = lax.GatherDimensionNumbers(
    offset_dims=(), collapsed_slice_dims=(0,), start_index_map=(0,))


def _perm(v, idx):
    return lax.gather(v, idx[:, None], _GDN, slice_sizes=(1,),
                      mode=lax.GatherScatterMode.PROMISE_IN_BOUNDS)


def _splat_min(v, lane):
    for sh in (8, 4, 2, 1):
        v = jnp.minimum(v, _perm(v, lane ^ sh))
    return v


def _splat_max(v, lane):
    for sh in (8, 4, 2, 1):
        v = jnp.maximum(v, _perm(v, lane ^ sh))
    return v


def _tree8(vals, op):
    return op(op(op(vals[0], vals[1]), op(vals[2], vals[3])),
              op(op(vals[4], vals[5]), op(vals[6], vals[7])))


def _dist_body(pt_ref, ps_ref, d2_ref, aux_ref):
    t0 = pt_ref[:, 0:1]
    t1 = pt_ref[:, 1:2]
    t2 = pt_ref[:, 2:3]
    e0 = t0 - ps_ref[0:1, :]
    e1 = t1 - ps_ref[1:2, :]
    e2 = t2 - ps_ref[2:3, :]
    d2 = e0 * e0 + e1 * e1 + e2 * e2
    d2_ref[...] = d2
    aux_ref[...] = jnp.min(d2.reshape(TB, NCH, 128), axis=2)


def _dist_and_chunkmins(pt_pad, ps_pad, rows):
    return pl.pallas_call(
        _dist_body,
        grid=(rows // TB,),
        in_specs=[
            pl.BlockSpec((TB, 3), lambda i: (i, 0)),
            pl.BlockSpec((3, NP), lambda i: (0, 0)),
        ],
        out_specs=[
            pl.BlockSpec((TB, NP), lambda i: (i, 0)),
            pl.BlockSpec((TB, NCH), lambda i: (i, 0)),
        ],
        out_shape=[
            jax.ShapeDtypeStruct((rows, NP), jnp.float32),
            jax.ShapeDtypeStruct((rows, NCH), jnp.float32),
        ],
    )(pt_pad, ps_pad)


TBM = 64


def _max_body(g_ref, o_ref):
    o_ref[...] = jnp.max(g_ref[...], axis=1)


def _max_over_k(gath, rows):
    return pl.pallas_call(
        _max_body,
        grid=(rows // TBM,),
        in_specs=[pl.BlockSpec((TBM, K, D), lambda i: (i, 0, 0))],
        out_specs=pl.BlockSpec((TBM, D), lambda i: (i, 0)),
        out_shape=jax.ShapeDtypeStruct((rows, D), jnp.float32),
    )(gath)


_SC_CACHE = {}


def _get_sc_select_gather(rows):
    if rows in _SC_CACHE:
        return _SC_CACHE[rows]
    RW = rows // _NW

    @functools.partial(
        pl.kernel,
        out_type=jax.ShapeDtypeStruct((rows, K, D), jnp.float32),
        mesh=plsc.VectorSubcoreMesh(core_axis_name="c", subcore_axis_name="s"),
        scratch_types=[
            pltpu.VMEM((RW, NCH), jnp.float32),
            pltpu.VMEM((NP,), jnp.float32),
            pltpu.VMEM((NP,), jnp.float32),
            pltpu.VMEM((K, D), jnp.float32),
            pltpu.VMEM((K, D), jnp.float32),
            pltpu.SemaphoreType.DMA,
            pltpu.SemaphoreType.DMA,
            pltpu.SemaphoreType.DMA,
            pltpu.SemaphoreType.DMA,
            pltpu.SemaphoreType.DMA,
            pltpu.SemaphoreType.DMA,
        ],
    )
    def _sc_body(x_hbm, d2_hbm, aux_hbm, gath_hbm,
                 aux_v, d2a_v, d2b_v, rxa_v, rxb_v,
                 sema, semb, semxa, semxb, semoa, semob):
        wid = lax.axis_index("s") * _NCORES + lax.axis_index("c")
        base = wid * RW
        pltpu.sync_copy(aux_hbm.at[pl.ds(base, RW)], aux_v)
        lane = lax.iota(jnp.int32, 16)

        chid = [None] * NG

        def _round(st, j):
            r, d2r_v, mvs, sel = st
            macc = jnp.minimum(jnp.minimum(jnp.minimum(mvs[0], mvs[1]),
                                           jnp.minimum(mvs[2], mvs[3])),
                               mvs[4])
            msp = _splat_min(macc, lane)
            cacc = jnp.where(mvs[0] == msp, chid[0], _I32_BIG)
            for g in range(1, NG):
                cacc = jnp.minimum(
                    cacc, jnp.where(mvs[g] == msp, chid[g], _I32_BIG))
            csp = _splat_min(cacc, lane)
            cbase = csp[0] * 128
            cps, cxs, sms = [], [], []
            for s in range(8):
                dv = d2r_v[pl.ds(cbase + s * 16, 16)]
                cols = cbase + s * 16 + lane
                iseq = dv == msp
                cps.append(jnp.where(iseq, cols, _I32_BIG))
                cxs.append(jnp.where(iseq, cols, -1))
                sms.append(jnp.where(dv > msp, dv, _F32_INF))
            colsp = _splat_min(_tree8(cps, jnp.minimum), lane)
            cmxsp = _splat_max(_tree8(cxs, jnp.maximum), lane)
            smsp = _splat_min(_tree8(sms, jnp.minimum), lane)
            csel = colsp[0]
            sel = jnp.where(lane == j, colsp, sel)
            soff = (csel // 16) * 16
            dvv = d2r_v[pl.ds(soff, 16)]
            d2r_v[pl.ds(soff, 16)] = jnp.where(
                soff + lane == colsp, _F32_INF, dvv)
            nsp = jnp.where(cmxsp > colsp, msp, smsp)
            mvs = [jnp.where(chid[g] == csp, nsp, mvs[g])
                   for g in range(NG)]
            return (r, d2r_v, mvs, sel)

        def select_rows2(r0, d2e_v, r1, d2o_v):
            for g in range(NG):
                chid[g] = lane + g * 16
            sts = []
            for r, dref in ((r0, d2e_v), (r1, d2o_v)):
                mvs = [aux_v[r, pl.ds(g * 16, 16)] for g in range(NG)]
                sel = jnp.zeros((16,), jnp.int32)
                sts.append((r, dref, mvs, sel))
            for j in range(K):
                sts = [_round(st, j) for st in sts]
            return sts[0][3], sts[1][3]

        pltpu.async_copy(d2_hbm.at[base], d2a_v, sema)
        pltpu.async_copy(d2_hbm.at[base + 1], d2b_v, semb)

        def pair_body(i, carry):
            r0 = i * 2

            @pl.when(i > 0)
            def _():
                pltpu.make_async_copy(x_hbm.at[lane], rxa_v, semxa).wait()
                pltpu.async_copy(rxa_v, gath_hbm.at[base + r0 - 2], semoa)
                pltpu.make_async_copy(x_hbm.at[lane], rxb_v, semxb).wait()
                pltpu.async_copy(rxb_v, gath_hbm.at[base + r0 - 1], semob)

            pltpu.make_async_copy(d2_hbm.at[base], d2a_v, sema).wait()
            pltpu.make_async_copy(d2_hbm.at[base], d2b_v, semb).wait()
            sel_e, sel_o = select_rows2(r0, d2a_v, r0 + 1, d2b_v)

            @pl.when(r0 + 2 < RW)
            def _():
                pltpu.async_copy(d2_hbm.at[base + r0 + 2], d2a_v, sema)

            @pl.when(r0 + 3 < RW)
            def _():
                pltpu.async_copy(d2_hbm.at[base + r0 + 3], d2b_v, semb)

            @pl.when(i > 0)
            def _():
                pltpu.make_async_copy(rxa_v, gath_hbm.at[base], semoa).wait()
                pltpu.make_async_copy(rxb_v, gath_hbm.at[base], semob).wait()
            pltpu.async_copy(x_hbm.at[sel_e], rxa_v, semxa)
            pltpu.async_copy(x_hbm.at[sel_o], rxb_v, semxb)
            return carry

        lax.fori_loop(0, RW // 2, pair_body, 0)
        pltpu.make_async_copy(x_hbm.at[lane], rxa_v, semxa).wait()
        pltpu.sync_copy(rxa_v, gath_hbm.at[base + RW - 2])
        pltpu.make_async_copy(x_hbm.at[lane], rxb_v, semxb).wait()
        pltpu.sync_copy(rxb_v, gath_hbm.at[base + RW - 1])

    _SC_CACHE[rows] = _sc_body
    return _sc_body


def kernel(x, pos, batch_x, pos_target, batch_target, k):
    pt_pad = jnp.zeros((NTP, 3), jnp.float32).at[:NT].set(pos_target)
    ps_pad = jnp.full((3, NP), _PAD_POS, jnp.float32).at[:, :N].set(pos.T)
    outs = []
    r0 = 0
    for rows in SPLITS:
        d2h, aux = _dist_and_chunkmins(pt_pad[r0:r0 + rows], ps_pad, rows)
        gath = _get_sc_select_gather(rows)(x, d2h, aux)
        outs.append(_max_over_k(gath, rows))
        r0 += rows
    return jnp.concatenate(outs, axis=0)[:NT]
